# baseline (speedup 1.0000x reference)
import functools

import jax
import jax.numpy as jnp
from jax import lax
from jax.experimental import pallas as pl
from jax.experimental.pallas import tpu as pltpu
from jax.experimental.pallas import tpu_sc as plsc

B = 16384
V = 1000000
D = 16
NS = 26
DENSE = 13

TCB = 8192
SUPER = 8
TGRID = 16
UNITS = TGRID * SUPER * TCB

NW = 32
BPW = B // NW
IDX_COLS = 128
RPW = (BPW * NS) // IDX_COLS
CHUNK = 8
N_CHUNKS = RPW // CHUNK


def _tr_body(in_ref, out_ref):
    x = in_ref[...]
    parts = [x[:, k * TCB:(k + 1) * TCB].T for k in range(SUPER)]
    out_ref[...] = jnp.concatenate(parts, axis=1)


def _tc_repack_table(tT):
    return pl.pallas_call(
        _tr_body,
        grid=(TGRID,),
        in_specs=[pl.BlockSpec((16, SUPER * TCB), lambda i: (0, i))],
        out_specs=pl.BlockSpec((TCB, 128), lambda i: (i, 0)),
        out_shape=jax.ShapeDtypeStruct((TGRID * TCB, 128), jnp.float32),
    )(tT)


def _sc_gather(table, sparse):
    mesh = plsc.VectorSubcoreMesh(core_axis_name="c", subcore_axis_name="s")
    out_rows = NW * RPW

    @functools.partial(
        pl.kernel,
        mesh=mesh,
        out_type=jax.ShapeDtypeStruct((out_rows, IDX_COLS, D), jnp.float32),
        scratch_types=[
            pltpu.VMEM((NS, BPW), jnp.int32),
            pltpu.VMEM((RPW, IDX_COLS), jnp.int32),
            pltpu.VMEM((2, CHUNK, IDX_COLS, D), jnp.float32),
            pltpu.SemaphoreType.DMA,
            pltpu.SemaphoreType.DMA,
        ],
        compiler_params=pltpu.CompilerParams(
            use_tc_tiling_on_sc=False, needs_layout_passes=False),
    )
    def k(table_hbm, idx_hbm, out_hbm, slab_v, idx_v, rows_v, sem_g, sem_o):
        wid = lax.axis_index("s") * 2 + lax.axis_index("c")
        base_b = wid * BPW
        base_r = wid * RPW

        pltpu.sync_copy(idx_hbm.at[:, pl.ds(base_b, BPW)], slab_v)

        lanes = lax.iota(jnp.int32, 16)

        def transpose_step(t, carry):
            s = t // (BPW // 16)
            j = t % (BPW // 16)
            v = slab_v[s, pl.ds(j * 16, 16)]
            vals = ((v >> 16) << 16) | ((v & 8191) << 3) | ((v >> 13) & 7)
            pos = (j * 16 + lanes) * NS + s
            plsc.store_scatter(idx_v, [pos >> 7, pos & 127], vals)
            return carry

        lax.fori_loop(0, NS * (BPW // 16), transpose_step, 0)

        for c in range(N_CHUNKS):
            buf = c % 2
            r0 = c * CHUNK
            if c >= 2:
                pltpu.make_async_copy(
                    rows_v.at[buf],
                    out_hbm.at[pl.ds(base_r + (c - 2) * CHUNK, CHUNK)],
                    sem_o,
                ).wait()

            def fire(j, carry, buf=buf, r0=r0):
                pltpu.async_copy(
                    table_hbm.at[idx_v.at[r0 + j]], rows_v.at[buf, j], sem_g)
                return carry

            lax.fori_loop(0, CHUNK, fire, 0)
            pltpu.make_async_copy(
                out_hbm.at[pl.ds(base_r + r0, CHUNK)], rows_v.at[buf], sem_g,
            ).wait()
            pltpu.async_copy(
                rows_v.at[buf], out_hbm.at[pl.ds(base_r + r0, CHUNK)], sem_o)

        for c in (N_CHUNKS - 2, N_CHUNKS - 1):
            pltpu.make_async_copy(
                rows_v.at[c % 2],
                out_hbm.at[pl.ds(base_r + c * CHUNK, CHUNK)],
                sem_o,
            ).wait()

    return k(table, sparse)


def _dotT(x, w):
    return lax.dot_general(x, w, (((1,), (1,)), ((), ())),
                           preferred_element_type=jnp.float32)


def _mlp_body(emb_ref, den_ref, w0e, w0d, b0r, w1, b1r, w2, b2r, w3, b3r,
              w4, ww, blast_ref, out_ref):
    x = emb_ref[...]
    d = den_ref[...]
    h = _dotT(x, w0e[...]) + _dotT(d, w0d[...]) + b0r[...]
    h = jnp.maximum(h, 0.0)
    h = jnp.maximum(_dotT(h, w1[...]) + b1r[...], 0.0)
    h = jnp.maximum(_dotT(h, w2[...]) + b2r[...], 0.0)
    h = jnp.maximum(_dotT(h, w3[...]) + b3r[...], 0.0)
    y = _dotT(h, w4[...])
    wide = _dotT(d, ww[...])
    out_ref[...] = y + wide + blast_ref[0]


def _tc_mlp(emb, dense, w0e, w0d, b0, w1, b1, w2, b2, w3, b3, w4, ww, blast):
    BM = 2048
    grid = (B // BM,)

    def const(shape):
        return pl.BlockSpec(shape, lambda i: (0, 0))

    return pl.pallas_call(
        _mlp_body,
        grid=grid,
        in_specs=[
            pl.BlockSpec((BM, NS * D), lambda i: (i, 0)),
            pl.BlockSpec((BM, DENSE), lambda i: (i, 0)),
            const((512, NS * D)),
            const((512, DENSE)),
            const((1, 512)),
            const((256, 512)),
            const((1, 256)),
            const((128, 256)),
            const((1, 128)),
            const((32, 128)),
            const((1, 32)),
            const((1, 32)),
            const((1, DENSE)),
            pl.BlockSpec(memory_space=pltpu.SMEM),
        ],
        out_specs=pl.BlockSpec((BM, 1), lambda i: (i, 0)),
        out_shape=jax.ShapeDtypeStruct((B, 1), jnp.float32),
    )(emb, dense, w0e, w0d, b0, w1, b1, w2, b2, w3, b3, w4, ww, blast)


def kernel(dense_feature, sparse_feature, table, Ww, bw, W0, b0, W1, b1, W2,
           b2, W3, b3, W4, b4):
    table_rm = _tc_repack_table(table.T)
    table8 = table_rm.reshape(-1).reshape(UNITS, D)
    emb3d = _sc_gather(table8, sparse_feature)
    emb = emb3d.reshape(B, NS * D)
    out = _tc_mlp(
        emb, dense_feature,
        W0[:, : NS * D], W0[:, NS * D:], b0.reshape(1, -1),
        W1, b1.reshape(1, -1),
        W2, b2.reshape(1, -1),
        W3, b3.reshape(1, -1),
        W4, Ww, (b4 + bw).reshape(1),
    )
    return out


# speedup vs baseline: 1.2110x; 1.2110x over previous
import functools

import jax
import jax.numpy as jnp
from jax import lax
from jax.experimental import pallas as pl
from jax.experimental.pallas import tpu as pltpu
from jax.experimental.pallas import tpu_sc as plsc

B = 16384
V = 1000000
D = 16
NS = 26
DENSE = 13

TCB = 8192
TGRID = 123
VPAD = TGRID * TCB
UNITS = VPAD * 8

NW = 32
BPW = B // NW
CHUNK = 8
NPLANE = 4
EPW = NPLANE * BPW * 8
GCHUNKS = 16


def _tr_body(in_ref, out_ref):
    z = in_ref[...].T
    out_ref[...] = jnp.pad(z, ((0, 0), (0, 112)))


def _tc_repack_table(tT):
    return pl.pallas_call(
        _tr_body,
        grid=(TGRID,),
        in_specs=[pl.BlockSpec((16, TCB), lambda i: (0, i))],
        out_specs=pl.BlockSpec((TCB, 128), lambda i: (i, 0)),
        out_shape=jax.ShapeDtypeStruct((VPAD, 128), jnp.float32),
    )(tT)


def _sc_gather(table, sparse):
    mesh = plsc.VectorSubcoreMesh(core_axis_name="c", subcore_axis_name="s")

    @functools.partial(
        pl.kernel,
        mesh=mesh,
        out_type=jax.ShapeDtypeStruct((NPLANE * B // 16, 128, D), jnp.float32),
        scratch_types=[
            pltpu.VMEM((NS, BPW), jnp.int32),
            pltpu.VMEM((EPW // 128, 128), jnp.int32),
            pltpu.VMEM((CHUNK, 128, D), jnp.float32),
            pltpu.VMEM((CHUNK, 128, D), jnp.float32),
            pltpu.SemaphoreType.DMA,
            pltpu.SemaphoreType.DMA,
        ],
        compiler_params=pltpu.CompilerParams(
            use_tc_tiling_on_sc=False, needs_layout_passes=False),
    )
    def k(table_hbm, idx_hbm, out_hbm, slab_v, idx_v, rows_a, rows_b,
          sem_g, sem_o):
        wid = lax.axis_index("s") * 2 + lax.axis_index("c")
        base_b = wid * BPW

        pltpu.sync_copy(idx_hbm.at[:, pl.ds(base_b, BPW)], slab_v)

        lanes = lax.iota(jnp.int32, 16)

        def transpose_step(t, carry):
            s32 = t // (BPW // 16)
            j = t % (BPW // 16)
            s = jnp.minimum(s32, NS - 1)
            vals = slab_v[s, pl.ds(j * 16, 16)] << 3
            e = (s32 // 8) * (BPW * 8) + (j * 16 + lanes) * 8 + (s32 % 8)
            plsc.store_scatter(idx_v, [e >> 7, e & 127], vals)
            return carry

        lax.fori_loop(0, 32 * (BPW // 16), transpose_step, 0)

        def out_dst(c):
            row0 = (c // 4) * 1024 + wid * 32 + (c % 4) * CHUNK
            return out_hbm.at[pl.ds(row0, CHUNK)]

        bufs = (rows_a, rows_b)
        for c in range(GCHUNKS):
            buf = bufs[c % 2]
            r0 = c * CHUNK
            if c >= 2:
                pltpu.make_async_copy(buf, out_dst(c - 2), sem_o).wait()

            def fire(j, carry, buf=buf, r0=r0):
                pltpu.async_copy(
                    table_hbm.at[idx_v.at[r0 + j]], buf.at[j], sem_g)
                return carry

            lax.fori_loop(0, CHUNK, fire, 0)
            pltpu.make_async_copy(out_dst(c), buf, sem_g).wait()
            pltpu.async_copy(buf, out_dst(c), sem_o)

        for c in (GCHUNKS - 2, GCHUNKS - 1):
            pltpu.make_async_copy(bufs[c % 2], out_dst(c), sem_o).wait()

    return k(table, sparse)


def _dotT(x, w):
    return lax.dot_general(x, w, (((1,), (1,)), ((), ())),
                           preferred_element_type=jnp.float32)


def _mlp_body(emb_ref, den_ref, w0e, w0d, b0r, w1, b1r, w2, b2r, w3, b3r,
              w4, ww, blast_ref, out_ref):
    x4 = emb_ref[...]
    d = den_ref[...]
    w0p = w0e[...]
    h = _dotT(x4[0], w0p[:, :128])
    for t in range(1, NPLANE):
        h = h + _dotT(x4[t], w0p[:, t * 128:(t + 1) * 128])
    h = h + _dotT(d, w0d[...]) + b0r[...]
    h = jnp.maximum(h, 0.0)
    h = jnp.maximum(_dotT(h, w1[...]) + b1r[...], 0.0)
    h = jnp.maximum(_dotT(h, w2[...]) + b2r[...], 0.0)
    h = jnp.maximum(_dotT(h, w3[...]) + b3r[...], 0.0)
    y = _dotT(h, w4[...])
    wide = _dotT(d, ww[...])
    out_ref[...] = y + wide + blast_ref[0]


def _tc_mlp(emb, dense, w0e, w0d, b0, w1, b1, w2, b2, w3, b3, w4, ww, blast):
    BM = 2048
    grid = (B // BM,)

    def const(shape):
        return pl.BlockSpec(shape, lambda i: (0, 0))

    return pl.pallas_call(
        _mlp_body,
        grid=grid,
        in_specs=[
            pl.BlockSpec((NPLANE, BM, 128), lambda i: (0, i, 0)),
            pl.BlockSpec((BM, DENSE), lambda i: (i, 0)),
            const((512, NPLANE * 128)),
            const((512, DENSE)),
            const((1, 512)),
            const((256, 512)),
            const((1, 256)),
            const((128, 256)),
            const((1, 128)),
            const((32, 128)),
            const((1, 32)),
            const((1, 32)),
            const((1, DENSE)),
            pl.BlockSpec(memory_space=pltpu.SMEM),
        ],
        out_specs=pl.BlockSpec((BM, 1), lambda i: (i, 0)),
        out_shape=jax.ShapeDtypeStruct((B, 1), jnp.float32),
    )(emb, dense, w0e, w0d, b0, w1, b1, w2, b2, w3, b3, w4, ww, blast)


def kernel(dense_feature, sparse_feature, table, Ww, bw, W0, b0, W1, b1, W2,
           b2, W3, b3, W4, b4):
    table_rm = _tc_repack_table(table.T)
    table8 = table_rm.reshape(-1).reshape(UNITS, D)
    emb4 = _sc_gather(table8, sparse_feature).reshape(NPLANE, B, 128)
    w0p = jnp.pad(W0[:, : NS * D], ((0, 0), (0, NPLANE * 128 - NS * D)))
    out = _tc_mlp(
        emb4, dense_feature,
        w0p, W0[:, NS * D:], b0.reshape(1, -1),
        W1, b1.reshape(1, -1),
        W2, b2.reshape(1, -1),
        W3, b3.reshape(1, -1),
        W4, Ww, (b4 + bw).reshape(1),
    )
    return out


# speedup vs baseline: 1.2465x; 1.0293x over previous
import functools

import jax
import jax.numpy as jnp
from jax import lax
from jax.experimental import pallas as pl
from jax.experimental.pallas import tpu as pltpu
from jax.experimental.pallas import tpu_sc as plsc

B = 16384
V = 1000000
D = 16
NS = 26
DENSE = 13

TCB = 8192
TGRID = 62
VPAD = TGRID * TCB
UNITS = VPAD * 8

NW = 32
BPW = B // NW
CHUNK = 8
NPLANE = 4
EPW = NPLANE * BPW * 8
GCHUNKS = 16


def _tr_body(in_ref, out_ref):
    x = in_ref[...]
    z = jnp.concatenate([x[:, :TCB].T, x[:, TCB:].T], axis=1)
    out_ref[...] = jnp.pad(z, ((0, 0), (0, 96)))


def _tc_repack_table(tT):
    return pl.pallas_call(
        _tr_body,
        grid=(TGRID,),
        in_specs=[pl.BlockSpec((16, 2 * TCB), lambda i: (0, i))],
        out_specs=pl.BlockSpec((TCB, 128), lambda i: (i, 0)),
        out_shape=jax.ShapeDtypeStruct((VPAD, 128), jnp.float32),
    )(tT)


def _sc_gather(table, sparse):
    mesh = plsc.VectorSubcoreMesh(core_axis_name="c", subcore_axis_name="s")

    @functools.partial(
        pl.kernel,
        mesh=mesh,
        out_type=jax.ShapeDtypeStruct((NPLANE * B // 16, 128, D), jnp.float32),
        scratch_types=[
            pltpu.VMEM((NS, BPW), jnp.int32),
            pltpu.VMEM((EPW // 128, 128), jnp.int32),
            pltpu.VMEM((CHUNK, 128, D), jnp.float32),
            pltpu.VMEM((CHUNK, 128, D), jnp.float32),
            pltpu.SemaphoreType.DMA,
            pltpu.SemaphoreType.DMA,
        ],
        compiler_params=pltpu.CompilerParams(
            use_tc_tiling_on_sc=False, needs_layout_passes=False),
    )
    def k(table_hbm, idx_hbm, out_hbm, slab_v, idx_v, rows_a, rows_b,
          sem_g, sem_o):
        wid = lax.axis_index("s") * 2 + lax.axis_index("c")
        base_b = wid * BPW

        pltpu.sync_copy(idx_hbm.at[:, pl.ds(base_b, BPW)], slab_v)

        lanes = lax.iota(jnp.int32, 16)

        def transpose_step(t, carry):
            s32 = t // (BPW // 16)
            j = t % (BPW // 16)
            s = jnp.minimum(s32, NS - 1)
            v = slab_v[s, pl.ds(j * 16, 16)]
            vals = (((v >> 14) << 16) | ((v & 8191) << 3) | ((v >> 13) & 1))
            e = (s32 // 8) * (BPW * 8) + (j * 16 + lanes) * 8 + (s32 % 8)
            plsc.store_scatter(idx_v, [e >> 7, e & 127], vals)
            return carry

        lax.fori_loop(0, 32 * (BPW // 16), transpose_step, 0)

        def out_dst(c):
            row0 = (c // 4) * 1024 + wid * 32 + (c % 4) * CHUNK
            return out_hbm.at[pl.ds(row0, CHUNK)]

        bufs = (rows_a, rows_b)
        for c in range(GCHUNKS):
            buf = bufs[c % 2]
            r0 = c * CHUNK
            if c >= 2:
                pltpu.make_async_copy(buf, out_dst(c - 2), sem_o).wait()

            def fire(j, carry, buf=buf, r0=r0):
                pltpu.async_copy(
                    table_hbm.at[idx_v.at[r0 + j]], buf.at[j], sem_g)
                return carry

            lax.fori_loop(0, CHUNK, fire, 0)
            pltpu.make_async_copy(out_dst(c), buf, sem_g).wait()
            pltpu.async_copy(buf, out_dst(c), sem_o)

        for c in (GCHUNKS - 2, GCHUNKS - 1):
            pltpu.make_async_copy(bufs[c % 2], out_dst(c), sem_o).wait()

    return k(table, sparse)


def _dotT(x, w):
    return lax.dot_general(x, w, (((1,), (1,)), ((), ())),
                           preferred_element_type=jnp.float32)


def _mlp_body(emb_ref, den_ref, w0e, w0d, b0r, w1, b1r, w2, b2r, w3, b3r,
              w4, ww, blast_ref, out_ref):
    x4 = emb_ref[...]
    d = den_ref[...]
    w0p = w0e[...]
    h = _dotT(x4[0], w0p[:, :128])
    for t in range(1, NPLANE):
        h = h + _dotT(x4[t], w0p[:, t * 128:(t + 1) * 128])
    h = h + _dotT(d, w0d[...]) + b0r[...]
    h = jnp.maximum(h, 0.0)
    h = jnp.maximum(_dotT(h, w1[...]) + b1r[...], 0.0)
    h = jnp.maximum(_dotT(h, w2[...]) + b2r[...], 0.0)
    h = jnp.maximum(_dotT(h, w3[...]) + b3r[...], 0.0)
    y = _dotT(h, w4[...])
    wide = _dotT(d, ww[...])
    out_ref[...] = y + wide + blast_ref[0]


def _tc_mlp(emb, dense, w0e, w0d, b0, w1, b1, w2, b2, w3, b3, w4, ww, blast):
    BM = 2048
    grid = (B // BM,)

    def const(shape):
        return pl.BlockSpec(shape, lambda i: (0, 0))

    return pl.pallas_call(
        _mlp_body,
        grid=grid,
        in_specs=[
            pl.BlockSpec((NPLANE, BM, 128), lambda i: (0, i, 0)),
            pl.BlockSpec((BM, DENSE), lambda i: (i, 0)),
            const((512, NPLANE * 128)),
            const((512, DENSE)),
            const((1, 512)),
            const((256, 512)),
            const((1, 256)),
            const((128, 256)),
            const((1, 128)),
            const((32, 128)),
            const((1, 32)),
            const((1, 32)),
            const((1, DENSE)),
            pl.BlockSpec(memory_space=pltpu.SMEM),
        ],
        out_specs=pl.BlockSpec((BM, 1), lambda i: (i, 0)),
        out_shape=jax.ShapeDtypeStruct((B, 1), jnp.float32),
    )(emb, dense, w0e, w0d, b0, w1, b1, w2, b2, w3, b3, w4, ww, blast)


def kernel(dense_feature, sparse_feature, table, Ww, bw, W0, b0, W1, b1, W2,
           b2, W3, b3, W4, b4):
    table_rm = _tc_repack_table(table.T)
    table8 = table_rm.reshape(-1).reshape(UNITS, D)
    emb4 = _sc_gather(table8, sparse_feature).reshape(NPLANE, B, 128)
    w0p = jnp.pad(W0[:, : NS * D], ((0, 0), (0, NPLANE * 128 - NS * D)))
    out = _tc_mlp(
        emb4, dense_feature,
        w0p, W0[:, NS * D:], b0.reshape(1, -1),
        W1, b1.reshape(1, -1),
        W2, b2.reshape(1, -1),
        W3, b3.reshape(1, -1),
        W4, Ww, (b4 + bw).reshape(1),
    )
    return out
